# baseline (speedup 1.0000x reference)
import functools

import jax
import jax.numpy as jnp
from jax import lax
from jax.experimental import pallas as pl
from jax.experimental.pallas import tpu as pltpu
from jax.experimental.pallas import tpu_sc as plsc

H = 64
EPS = 1e-5
_NC, _NS = 2, 16
_NW = _NC * _NS


def _row_tile(n):
    for t in (2000, 1000, 500, 250):
        if n % t == 0:
            return t
    return n


def _embed_body(x_ref, w0_ref, b0_ref, w1_ref, b1_ref, out_ref, sums_ref, acc_ref):
    i = pl.program_id(0)

    @pl.when(i == 0)
    def _init():
        acc_ref[...] = jnp.zeros_like(acc_ref)

    h = jnp.maximum(jnp.dot(x_ref[...], w0_ref[...],
                            preferred_element_type=jnp.float32) + b0_ref[...], 0.0)
    y = jnp.maximum(jnp.dot(h, w1_ref[...],
                            preferred_element_type=jnp.float32) + b1_ref[...], 0.0)
    out_ref[...] = y
    s = jnp.sum(y, axis=0, keepdims=True)
    sq = jnp.sum(y * y, axis=0, keepdims=True)
    acc_ref[...] += jnp.concatenate([s, sq], axis=0)

    @pl.when(i == pl.num_programs(0) - 1)
    def _fin():
        sums_ref[...] = acc_ref[...]


def _embed(x, w0, b0, w1, b1):
    n, d = x.shape
    tile = _row_tile(n)
    grid = n // tile
    out, sums = pl.pallas_call(
        _embed_body,
        grid=(grid,),
        in_specs=[
            pl.BlockSpec((tile, d), lambda i: (i, 0)),
            pl.BlockSpec((d, H), lambda i: (0, 0)),
            pl.BlockSpec((1, H), lambda i: (0, 0)),
            pl.BlockSpec((H, H), lambda i: (0, 0)),
            pl.BlockSpec((1, H), lambda i: (0, 0)),
        ],
        out_specs=[
            pl.BlockSpec((tile, H), lambda i: (i, 0)),
            pl.BlockSpec((2, H), lambda i: (0, 0)),
        ],
        out_shape=[
            jax.ShapeDtypeStruct((n, H), jnp.float32),
            jax.ShapeDtypeStruct((2, H), jnp.float32),
        ],
        scratch_shapes=[pltpu.VMEM((2, H), jnp.float32)],
    )(x, w0, b0.reshape(1, H), w1, b1.reshape(1, H))
    return out, sums


def _mm_body(x_ref, w_ref, out_ref):
    out_ref[...] = jnp.dot(x_ref[...], w_ref[...], preferred_element_type=jnp.float32)


def _mm(x, w):
    n = x.shape[0]
    m = w.shape[1]
    tile = _row_tile(n)
    return pl.pallas_call(
        _mm_body,
        grid=(n // tile,),
        in_specs=[
            pl.BlockSpec((tile, H), lambda i: (i, 0)),
            pl.BlockSpec((H, m), lambda i: (0, 0)),
        ],
        out_specs=pl.BlockSpec((tile, m), lambda i: (i, 0)),
        out_shape=jax.ShapeDtypeStruct((n, m), jnp.float32),
    )(x, w)


def _edge_body(a_off, b_off, au_ref, av_ref, ev_ref, we_ref, b1_ref, w_ref, b2_ref,
               out_ref):
    hpre = (au_ref[:, a_off:a_off + H] + av_ref[:, b_off:b_off + H]
            + ev_ref[...] * we_ref[...] + b1_ref[...])
    h = jnp.maximum(hpre, 0.0)
    g = jnp.dot(h, w_ref[...], preferred_element_type=jnp.float32) + b2_ref[...]
    out_ref[...] = jnp.maximum(g, 0.0)


def _edge(au_g, a_off, av_g, b_off, e_val, we, b1, w, b2):
    e = au_g.shape[0]
    tile = 6400
    assert e % tile == 0
    return pl.pallas_call(
        functools.partial(_edge_body, a_off, b_off),
        grid=(e // tile,),
        in_specs=[
            pl.BlockSpec((tile, au_g.shape[1]), lambda i: (i, 0)),
            pl.BlockSpec((tile, av_g.shape[1]), lambda i: (i, 0)),
            pl.BlockSpec((tile, 1), lambda i: (i, 0)),
            pl.BlockSpec((1, H), lambda i: (0, 0)),
            pl.BlockSpec((1, H), lambda i: (0, 0)),
            pl.BlockSpec((H, H), lambda i: (0, 0)),
            pl.BlockSpec((1, H), lambda i: (0, 0)),
        ],
        out_specs=pl.BlockSpec((tile, H), lambda i: (i, 0)),
        out_shape=jax.ShapeDtypeStruct((e, H), jnp.float32),
    )(au_g, av_g, e_val, we.reshape(1, H), b1.reshape(1, H), w, b2.reshape(1, H))


def _fmlp_body(u_ref, agg_ref, wa_ref, wb_ref, b0_ref, w1_ref, b1_ref,
               out_ref, sums_ref, acc_ref):
    i = pl.program_id(0)

    @pl.when(i == 0)
    def _init():
        acc_ref[...] = jnp.zeros_like(acc_ref)

    h = jnp.dot(u_ref[...], wa_ref[...], preferred_element_type=jnp.float32)
    h += jnp.dot(agg_ref[...], wb_ref[...], preferred_element_type=jnp.float32)
    h = jnp.maximum(h + b0_ref[...], 0.0)
    y = jnp.maximum(jnp.dot(h, w1_ref[...],
                            preferred_element_type=jnp.float32) + b1_ref[...], 0.0)
    out_ref[...] = y
    s = jnp.sum(y, axis=0, keepdims=True)
    sq = jnp.sum(y * y, axis=0, keepdims=True)
    acc_ref[...] += jnp.concatenate([s, sq], axis=0)

    @pl.when(i == pl.num_programs(0) - 1)
    def _fin():
        sums_ref[...] = acc_ref[...]


def _fmlp(u, agg, wa, wb, b0, w1, b1):
    n = u.shape[0]
    tile = _row_tile(n)
    out, sums = pl.pallas_call(
        _fmlp_body,
        grid=(n // tile,),
        in_specs=[
            pl.BlockSpec((tile, H), lambda i: (i, 0)),
            pl.BlockSpec((tile, H), lambda i: (i, 0)),
            pl.BlockSpec((H, H), lambda i: (0, 0)),
            pl.BlockSpec((H, H), lambda i: (0, 0)),
            pl.BlockSpec((1, H), lambda i: (0, 0)),
            pl.BlockSpec((H, H), lambda i: (0, 0)),
            pl.BlockSpec((1, H), lambda i: (0, 0)),
        ],
        out_specs=[
            pl.BlockSpec((tile, H), lambda i: (i, 0)),
            pl.BlockSpec((2, H), lambda i: (0, 0)),
        ],
        out_shape=[
            jax.ShapeDtypeStruct((n, H), jnp.float32),
            jax.ShapeDtypeStruct((2, H), jnp.float32),
        ],
        scratch_shapes=[pltpu.VMEM((2, H), jnp.float32)],
    )(u, agg, wa, wb, b0.reshape(1, H), w1, b1.reshape(1, H))
    return out, sums


def _tail_body(x_ref, w0_ref, b0_ref, w1_ref, b1_ref, out_ref):
    h = jnp.maximum(jnp.dot(x_ref[...], w0_ref[...],
                            preferred_element_type=jnp.float32) + b0_ref[...], 0.0)
    o = jnp.dot(h, w1_ref[...], preferred_element_type=jnp.float32) + b1_ref[...]
    out_ref[...] = jax.nn.sigmoid(o)


def _tail(x, w0, b0, w1, b1):
    n = x.shape[0]
    k = w1.shape[1]
    tile = _row_tile(n)
    return pl.pallas_call(
        _tail_body,
        grid=(n // tile,),
        in_specs=[
            pl.BlockSpec((tile, H), lambda i: (i, 0)),
            pl.BlockSpec((H, H), lambda i: (0, 0)),
            pl.BlockSpec((1, H), lambda i: (0, 0)),
            pl.BlockSpec((H, k), lambda i: (0, 0)),
            pl.BlockSpec((1, k), lambda i: (0, 0)),
        ],
        out_specs=pl.BlockSpec((tile, k), lambda i: (i, 0)),
        out_shape=jax.ShapeDtypeStruct((n, k), jnp.float32),
    )(x, w0, b0.reshape(1, H), w1, b1.reshape(1, k))


def _sc_gather(table, idx):
    e = idx.shape[0]
    w128 = table.shape[1]
    b = 128
    epw = e // _NW
    assert e % _NW == 0 and epw % 8 == 0
    nchunks = (epw + b - 1) // b
    nchunks += nchunks % 2
    npairs = nchunks // 2
    mesh = plsc.VectorSubcoreMesh(core_axis_name="c", subcore_axis_name="s")

    @functools.partial(
        pl.kernel,
        out_type=jax.ShapeDtypeStruct((e, w128), jnp.float32),
        mesh=mesh,
        scratch_types=[
            pltpu.VMEM((2, b), jnp.int32),
            pltpu.VMEM((2, b, w128), jnp.float32),
            pltpu.SemaphoreType.DMA, pltpu.SemaphoreType.DMA,
        ],
    )
    def k(tab_hbm, idx_hbm, out_hbm, idx_v, buf, sem0, sem1):
        wid = lax.axis_index("s") * _NC + lax.axis_index("c")
        base = wid * epw

        def off_of(t):
            return base + jnp.minimum(t * b, epw - b)

        def start(t, slot, sem):
            pltpu.sync_copy(idx_hbm.at[pl.ds(off_of(t), b)], idx_v.at[slot])
            pltpu.async_copy(tab_hbm.at[idx_v.at[slot]], buf.at[slot], sem)

        def drain(t, slot, sem):
            pltpu.make_async_copy(tab_hbm.at[idx_v.at[slot]], buf.at[slot], sem).wait()
            pltpu.sync_copy(buf.at[slot], out_hbm.at[pl.ds(off_of(t), b)])

        start(0, 0, sem0)

        def body(i, carry):
            t0 = i * 2
            start(t0 + 1, 1, sem1)
            drain(t0, 0, sem0)

            @pl.when(i + 1 < npairs)
            def _():
                start(t0 + 2, 0, sem0)

            drain(t0 + 1, 1, sem1)
            return carry

        lax.fori_loop(0, npairs, body, 0)

    return k(table, idx)


def _sc_gather2(tab_a, idx_a, tab_b, idx_b):
    e = idx_a.shape[0]
    wa, wb = tab_a.shape[1], tab_b.shape[1]
    b = 128
    epw = e // _NW
    assert e % _NW == 0 and epw % 8 == 0
    nchunks = (epw + b - 1) // b
    nchunks += nchunks % 2
    npairs = nchunks // 2
    mesh = plsc.VectorSubcoreMesh(core_axis_name="c", subcore_axis_name="s")

    @functools.partial(
        pl.kernel,
        out_type=[jax.ShapeDtypeStruct((e, wa), jnp.float32),
                  jax.ShapeDtypeStruct((e, wb), jnp.float32)],
        mesh=mesh,
        scratch_types=[
            pltpu.VMEM((2, b), jnp.int32), pltpu.VMEM((2, b), jnp.int32),
            pltpu.VMEM((2, b, wa), jnp.float32), pltpu.VMEM((2, b, wb), jnp.float32),
            pltpu.SemaphoreType.DMA, pltpu.SemaphoreType.DMA,
            pltpu.SemaphoreType.DMA, pltpu.SemaphoreType.DMA,
        ],
    )
    def k(ta_hbm, ia_hbm, tb_hbm, ib_hbm, oa_hbm, ob_hbm,
          ia_v, ib_v, bufa, bufb, sa0, sa1, sb0, sb1):
        wid = lax.axis_index("s") * _NC + lax.axis_index("c")
        base = wid * epw

        def off_of(t):
            return base + jnp.minimum(t * b, epw - b)

        def start(t, slot, sema, semb):
            off = off_of(t)
            pltpu.sync_copy(ia_hbm.at[pl.ds(off, b)], ia_v.at[slot])
            pltpu.sync_copy(ib_hbm.at[pl.ds(off, b)], ib_v.at[slot])
            pltpu.async_copy(ta_hbm.at[ia_v.at[slot]], bufa.at[slot], sema)
            pltpu.async_copy(tb_hbm.at[ib_v.at[slot]], bufb.at[slot], semb)

        def drain(t, slot, sema, semb):
            off = off_of(t)
            pltpu.make_async_copy(ta_hbm.at[ia_v.at[slot]], bufa.at[slot], sema).wait()
            pltpu.sync_copy(bufa.at[slot], oa_hbm.at[pl.ds(off, b)])
            pltpu.make_async_copy(tb_hbm.at[ib_v.at[slot]], bufb.at[slot], semb).wait()
            pltpu.sync_copy(bufb.at[slot], ob_hbm.at[pl.ds(off, b)])

        start(0, 0, sa0, sb0)

        def body(i, carry):
            t0 = i * 2
            start(t0 + 1, 1, sa1, sb1)
            drain(t0, 0, sa0, sb0)

            @pl.when(i + 1 < npairs)
            def _():
                start(t0 + 2, 0, sa0, sb0)

            drain(t0 + 1, 1, sa1, sb1)
            return carry

        lax.fori_loop(0, npairs, body, 0)

    return k(tab_a, idx_a, tab_b, idx_b)


def _sc_segsum(vals, idx, n_out, n_ranges):
    e = vals.shape[0]
    rsize = -(-n_out // n_ranges)
    rsize = -(-rsize // 128) * 128
    n_pad = rsize * n_ranges
    ncks = rsize // 128
    nz = -(-ncks // _NS)
    b = 80
    ept = e // _NS
    assert e % _NS == 0 and ept % b == 0
    nchunks = ept // b
    rpc = n_ranges // 2
    mesh = plsc.VectorSubcoreMesh(core_axis_name="c", subcore_axis_name="s")

    @functools.partial(
        pl.kernel,
        out_type=[jax.ShapeDtypeStruct((rsize + 128, H), jnp.float32)
                  for _ in range(n_ranges)],
        mesh=mesh,
        scratch_types=[
            pltpu.VMEM((b,), jnp.int32),
            pltpu.VMEM((b,), jnp.int32),
            pltpu.VMEM((b, H), jnp.float32),
            pltpu.VMEM_SHARED((rsize + 128, H), jnp.float32),
        ],
    )
    def k(vals_hbm, idx_hbm, zeros_hbm, *rest):
        outs, (iraw, iadj, vbuf, acc) = rest[:n_ranges], rest[n_ranges:]
        cid = lax.axis_index("c")
        sid = lax.axis_index("s")
        ebase = sid * ept
        spread = lax.iota(jnp.int32, 16) * 8

        for r in range(rpc):
            @pl.when(sid == 0)
            def _zinit():
                pltpu.sync_copy(zeros_hbm, acc)

            plsc.subcore_barrier()
            nbase_t = (cid * rpc + r) * rsize

            def scan(t, carry):
                off = ebase + t * b
                pltpu.sync_copy(idx_hbm.at[pl.ds(off, b)], iraw)
                pltpu.sync_copy(vals_hbm.at[pl.ds(off, b)], vbuf)
                for j in range(b // 16):
                    x = iraw[pl.ds(j * 16, 16)] - nbase_t
                    oob = (x < 0) | (x >= rsize)
                    iadj[pl.ds(j * 16, 16)] = jnp.where(oob, rsize + spread + j, x)
                pltpu.sync_copy(vbuf, acc.at[iadj], add=True)
                return carry

            lax.fori_loop(0, nchunks, scan, 0)
            plsc.subcore_barrier()

            @pl.when((sid == 0) & (cid == 0))
            def _w0():
                pltpu.sync_copy(acc, outs[r])

            @pl.when((sid == 0) & (cid == 1))
            def _w1():
                pltpu.sync_copy(acc, outs[rpc + r])

            plsc.subcore_barrier()

    outs = k(vals, idx, jnp.zeros((rsize + 128, H), jnp.float32))
    return jnp.concatenate([o[:rsize] for o in outs], axis=0)


def _fold_bn(sums, n, gamma, beta):
    mean = sums[0] / n
    var = sums[1] / n - mean * mean
    s = gamma * lax.rsqrt(var + EPS)
    t = beta - mean * s
    return s, t


def kernel(v, c, e_row, e_col, e_val,
           emb_v_W0, emb_v_b0, emb_v_W1, emb_v_b1,
           emb_c_W0, emb_c_b0, emb_c_W1, emb_c_b1,
           bn_v_gamma, bn_v_beta, bn_c_gamma, bn_c_beta,
           cconv_g_W0, cconv_g_b0, cconv_g_W1, cconv_g_b1,
           cconv_f_W0, cconv_f_b0, cconv_f_W1, cconv_f_b1,
           bn_c2_gamma, bn_c2_beta,
           vconv_g_W0, vconv_g_b0, vconv_g_W1, vconv_g_b1,
           vconv_f_W0, vconv_f_b0, vconv_f_W1, vconv_f_b1,
           bn_v2_gamma, bn_v2_beta,
           tail_W0, tail_b0, tail_W1, tail_b1):
    v_n = v.shape[0]
    c_n = c.shape[0]

    v1r, v_sums = _embed(v, emb_v_W0, emb_v_b0, emb_v_W1, emb_v_b1)
    c1r, c_sums = _embed(c, emb_c_W0, emb_c_b0, emb_c_W1, emb_c_b1)
    s_v, t_v = _fold_bn(v_sums, v_n, bn_v_gamma, bn_v_beta)
    s_c, t_c = _fold_bn(c_sums, c_n, bn_c_gamma, bn_c_beta)

    wu, wv, we = cconv_g_W0[:H], cconv_g_W0[H:2 * H], cconv_g_W0[2 * H]
    wu2, wv2, we2 = vconv_g_W0[:H], vconv_g_W0[H:2 * H], vconv_g_W0[2 * H]
    tab_v = _mm(v1r, jnp.concatenate(
        [s_v[:, None] * wv, s_v[:, None] * wu2], axis=1))
    tab_c = _mm(c1r, jnp.concatenate(
        [s_c[:, None] * wu, jnp.zeros_like(wu)], axis=1))
    gv, gc = _sc_gather2(tab_v, e_row, tab_c, e_col)
    b1 = t_c @ wu + t_v @ wv + cconv_g_b0
    g_out = _edge(gc, 0, gv, 0, e_val, we, b1, cconv_g_W1, cconv_g_b1)
    agg_c = jnp.zeros((c_n, H), jnp.float32).at[e_col].add(g_out, mode='promise_in_bounds')
    wfa, wfb = cconv_f_W0[:H], cconv_f_W0[H:]
    c2r, c2_sums = _fmlp(c1r, agg_c, s_c[:, None] * wfa, wfb,
                         t_c @ wfa + cconv_f_b0, cconv_f_W1, cconv_f_b1)
    s_c2, t_c2 = _fold_bn(c2_sums, c_n, bn_c2_gamma, bn_c2_beta)

    tab_c2 = _mm(c2r, jnp.concatenate(
        [s_c2[:, None] * wv2, jnp.zeros_like(wv2)], axis=1))
    gc2 = _sc_gather(tab_c2, e_col)
    b12 = t_v @ wu2 + t_c2 @ wv2 + vconv_g_b0
    g_out2 = _edge(gv, H, gc2, 0, e_val, we2, b12, vconv_g_W1, vconv_g_b1)
    agg_v = jnp.zeros((v_n, H), jnp.float32).at[e_row].add(g_out2, mode='promise_in_bounds')
    wfa2, wfb2 = vconv_f_W0[:H], vconv_f_W0[H:]
    v2r, v2_sums = _fmlp(v1r, agg_v, s_v[:, None] * wfa2, wfb2,
                         t_v @ wfa2 + vconv_f_b0, vconv_f_W1, vconv_f_b1)
    s_v2, t_v2 = _fold_bn(v2_sums, v_n, bn_v2_gamma, bn_v2_beta)

    return _tail(v2r, s_v2[:, None] * tail_W0, t_v2 @ tail_W0 + tail_b0,
                 tail_W1, tail_b1)


# speedup vs baseline: 1.2758x; 1.2758x over previous
import functools

import jax
import jax.numpy as jnp
from jax import lax
from jax.experimental import pallas as pl
from jax.experimental.pallas import tpu as pltpu
from jax.experimental.pallas import tpu_sc as plsc

H = 64
EPS = 1e-5
_NC, _NS = 2, 16
_NW = _NC * _NS


def _row_tile(n):
    for t in (2000, 1000, 500, 250):
        if n % t == 0:
            return t
    return n


def _embed_body(x_ref, w0_ref, b0_ref, w1_ref, b1_ref, out_ref, sums_ref, acc_ref):
    i = pl.program_id(0)

    @pl.when(i == 0)
    def _init():
        acc_ref[...] = jnp.zeros_like(acc_ref)

    h = jnp.maximum(jnp.dot(x_ref[...], w0_ref[...],
                            preferred_element_type=jnp.float32) + b0_ref[...], 0.0)
    y = jnp.maximum(jnp.dot(h, w1_ref[...],
                            preferred_element_type=jnp.float32) + b1_ref[...], 0.0)
    out_ref[...] = y
    s = jnp.sum(y, axis=0, keepdims=True)
    sq = jnp.sum(y * y, axis=0, keepdims=True)
    acc_ref[...] += jnp.concatenate([s, sq], axis=0)

    @pl.when(i == pl.num_programs(0) - 1)
    def _fin():
        sums_ref[...] = acc_ref[...]


def _embed(x, w0, b0, w1, b1):
    n, d = x.shape
    tile = _row_tile(n)
    grid = n // tile
    out, sums = pl.pallas_call(
        _embed_body,
        grid=(grid,),
        in_specs=[
            pl.BlockSpec((tile, d), lambda i: (i, 0)),
            pl.BlockSpec((d, H), lambda i: (0, 0)),
            pl.BlockSpec((1, H), lambda i: (0, 0)),
            pl.BlockSpec((H, H), lambda i: (0, 0)),
            pl.BlockSpec((1, H), lambda i: (0, 0)),
        ],
        out_specs=[
            pl.BlockSpec((tile, H), lambda i: (i, 0)),
            pl.BlockSpec((2, H), lambda i: (0, 0)),
        ],
        out_shape=[
            jax.ShapeDtypeStruct((n, H), jnp.float32),
            jax.ShapeDtypeStruct((2, H), jnp.float32),
        ],
        scratch_shapes=[pltpu.VMEM((2, H), jnp.float32)],
    )(x, w0, b0.reshape(1, H), w1, b1.reshape(1, H))
    return out, sums


def _mm_body(x_ref, w_ref, out_ref):
    out_ref[...] = jnp.dot(x_ref[...], w_ref[...], preferred_element_type=jnp.float32)


def _mm(x, w):
    n = x.shape[0]
    m = w.shape[1]
    tile = _row_tile(n)
    return pl.pallas_call(
        _mm_body,
        grid=(n // tile,),
        in_specs=[
            pl.BlockSpec((tile, H), lambda i: (i, 0)),
            pl.BlockSpec((H, m), lambda i: (0, 0)),
        ],
        out_specs=pl.BlockSpec((tile, m), lambda i: (i, 0)),
        out_shape=jax.ShapeDtypeStruct((n, m), jnp.float32),
    )(x, w)


def _edge_body(a_off, b_off, au_ref, av_ref, ev_ref, we_ref, b1_ref, w_ref, b2_ref,
               out_ref):
    hpre = (au_ref[:, a_off:a_off + H] + av_ref[:, b_off:b_off + H]
            + ev_ref[...] * we_ref[...] + b1_ref[...])
    h = jnp.maximum(hpre, 0.0)
    g = jnp.dot(h, w_ref[...], preferred_element_type=jnp.float32) + b2_ref[...]
    out_ref[...] = jnp.maximum(g, 0.0)


def _edge(au_g, a_off, av_g, b_off, e_val, we, b1, w, b2):
    e = au_g.shape[0]
    tile = 6400
    assert e % tile == 0
    return pl.pallas_call(
        functools.partial(_edge_body, a_off, b_off),
        grid=(e // tile,),
        in_specs=[
            pl.BlockSpec((tile, au_g.shape[1]), lambda i: (i, 0)),
            pl.BlockSpec((tile, av_g.shape[1]), lambda i: (i, 0)),
            pl.BlockSpec((tile, 1), lambda i: (i, 0)),
            pl.BlockSpec((1, H), lambda i: (0, 0)),
            pl.BlockSpec((1, H), lambda i: (0, 0)),
            pl.BlockSpec((H, H), lambda i: (0, 0)),
            pl.BlockSpec((1, H), lambda i: (0, 0)),
        ],
        out_specs=pl.BlockSpec((tile, H), lambda i: (i, 0)),
        out_shape=jax.ShapeDtypeStruct((e, H), jnp.float32),
    )(au_g, av_g, e_val, we.reshape(1, H), b1.reshape(1, H), w, b2.reshape(1, H))


def _fmlp_body(u_ref, agg_ref, wa_ref, wb_ref, b0_ref, w1_ref, b1_ref,
               out_ref, sums_ref, acc_ref):
    i = pl.program_id(0)

    @pl.when(i == 0)
    def _init():
        acc_ref[...] = jnp.zeros_like(acc_ref)

    h = jnp.dot(u_ref[...], wa_ref[...], preferred_element_type=jnp.float32)
    h += jnp.dot(agg_ref[...], wb_ref[...], preferred_element_type=jnp.float32)
    h = jnp.maximum(h + b0_ref[...], 0.0)
    y = jnp.maximum(jnp.dot(h, w1_ref[...],
                            preferred_element_type=jnp.float32) + b1_ref[...], 0.0)
    out_ref[...] = y
    s = jnp.sum(y, axis=0, keepdims=True)
    sq = jnp.sum(y * y, axis=0, keepdims=True)
    acc_ref[...] += jnp.concatenate([s, sq], axis=0)

    @pl.when(i == pl.num_programs(0) - 1)
    def _fin():
        sums_ref[...] = acc_ref[...]


def _fmlp(u, agg, wa, wb, b0, w1, b1):
    n = u.shape[0]
    tile = _row_tile(n)
    out, sums = pl.pallas_call(
        _fmlp_body,
        grid=(n // tile,),
        in_specs=[
            pl.BlockSpec((tile, H), lambda i: (i, 0)),
            pl.BlockSpec((tile, H), lambda i: (i, 0)),
            pl.BlockSpec((H, H), lambda i: (0, 0)),
            pl.BlockSpec((H, H), lambda i: (0, 0)),
            pl.BlockSpec((1, H), lambda i: (0, 0)),
            pl.BlockSpec((H, H), lambda i: (0, 0)),
            pl.BlockSpec((1, H), lambda i: (0, 0)),
        ],
        out_specs=[
            pl.BlockSpec((tile, H), lambda i: (i, 0)),
            pl.BlockSpec((2, H), lambda i: (0, 0)),
        ],
        out_shape=[
            jax.ShapeDtypeStruct((n, H), jnp.float32),
            jax.ShapeDtypeStruct((2, H), jnp.float32),
        ],
        scratch_shapes=[pltpu.VMEM((2, H), jnp.float32)],
    )(u, agg, wa, wb, b0.reshape(1, H), w1, b1.reshape(1, H))
    return out, sums


def _tail_body(x_ref, w0_ref, b0_ref, w1_ref, b1_ref, out_ref):
    h = jnp.maximum(jnp.dot(x_ref[...], w0_ref[...],
                            preferred_element_type=jnp.float32) + b0_ref[...], 0.0)
    o = jnp.dot(h, w1_ref[...], preferred_element_type=jnp.float32) + b1_ref[...]
    out_ref[...] = jax.nn.sigmoid(o)


def _tail(x, w0, b0, w1, b1):
    n = x.shape[0]
    k = w1.shape[1]
    tile = _row_tile(n)
    return pl.pallas_call(
        _tail_body,
        grid=(n // tile,),
        in_specs=[
            pl.BlockSpec((tile, H), lambda i: (i, 0)),
            pl.BlockSpec((H, H), lambda i: (0, 0)),
            pl.BlockSpec((1, H), lambda i: (0, 0)),
            pl.BlockSpec((H, k), lambda i: (0, 0)),
            pl.BlockSpec((1, k), lambda i: (0, 0)),
        ],
        out_specs=pl.BlockSpec((tile, k), lambda i: (i, 0)),
        out_shape=jax.ShapeDtypeStruct((n, k), jnp.float32),
    )(x, w0, b0.reshape(1, H), w1, b1.reshape(1, k))


def _sc_gather(table, idx):
    e = idx.shape[0]
    w128 = table.shape[1]
    b = 128
    epw = e // _NW
    assert e % _NW == 0 and epw % 8 == 0
    nchunks = (epw + b - 1) // b
    nchunks += nchunks % 2
    npairs = nchunks // 2
    mesh = plsc.VectorSubcoreMesh(core_axis_name="c", subcore_axis_name="s")

    @functools.partial(
        pl.kernel,
        out_type=jax.ShapeDtypeStruct((e, w128), jnp.float32),
        mesh=mesh,
        scratch_types=[
            pltpu.VMEM((2, b), jnp.int32),
            pltpu.VMEM((2, b, w128), jnp.float32),
            pltpu.SemaphoreType.DMA, pltpu.SemaphoreType.DMA,
        ],
    )
    def k(tab_hbm, idx_hbm, out_hbm, idx_v, buf, sem0, sem1):
        wid = lax.axis_index("s") * _NC + lax.axis_index("c")
        base = wid * epw

        def off_of(t):
            return base + jnp.minimum(t * b, epw - b)

        def start(t, slot, sem):
            pltpu.sync_copy(idx_hbm.at[pl.ds(off_of(t), b)], idx_v.at[slot])
            pltpu.async_copy(tab_hbm.at[idx_v.at[slot]], buf.at[slot], sem)

        def drain(t, slot, sem):
            pltpu.make_async_copy(tab_hbm.at[idx_v.at[slot]], buf.at[slot], sem).wait()
            pltpu.sync_copy(buf.at[slot], out_hbm.at[pl.ds(off_of(t), b)])

        start(0, 0, sem0)

        def body(i, carry):
            t0 = i * 2
            start(t0 + 1, 1, sem1)
            drain(t0, 0, sem0)

            @pl.when(i + 1 < npairs)
            def _():
                start(t0 + 2, 0, sem0)

            drain(t0 + 1, 1, sem1)
            return carry

        lax.fori_loop(0, npairs, body, 0)

    return k(table, idx)


def _sc_gather2(tab_a, idx_a, tab_b, idx_b):
    e = idx_a.shape[0]
    wa, wb = tab_a.shape[1], tab_b.shape[1]
    b = 128
    epw = e // _NW
    assert e % _NW == 0 and epw % 8 == 0
    nchunks = (epw + b - 1) // b
    nchunks += nchunks % 2
    npairs = nchunks // 2
    mesh = plsc.VectorSubcoreMesh(core_axis_name="c", subcore_axis_name="s")

    @functools.partial(
        pl.kernel,
        out_type=[jax.ShapeDtypeStruct((e, wa), jnp.float32),
                  jax.ShapeDtypeStruct((e, wb), jnp.float32)],
        mesh=mesh,
        scratch_types=[
            pltpu.VMEM((2, b), jnp.int32), pltpu.VMEM((2, b), jnp.int32),
            pltpu.VMEM((2, b, wa), jnp.float32), pltpu.VMEM((2, b, wb), jnp.float32),
            pltpu.SemaphoreType.DMA, pltpu.SemaphoreType.DMA,
            pltpu.SemaphoreType.DMA, pltpu.SemaphoreType.DMA,
        ],
    )
    def k(ta_hbm, ia_hbm, tb_hbm, ib_hbm, oa_hbm, ob_hbm,
          ia_v, ib_v, bufa, bufb, sa0, sa1, sb0, sb1):
        wid = lax.axis_index("s") * _NC + lax.axis_index("c")
        base = wid * epw

        def off_of(t):
            return base + jnp.minimum(t * b, epw - b)

        def start(t, slot, sema, semb):
            off = off_of(t)
            pltpu.sync_copy(ia_hbm.at[pl.ds(off, b)], ia_v.at[slot])
            pltpu.sync_copy(ib_hbm.at[pl.ds(off, b)], ib_v.at[slot])
            pltpu.async_copy(ta_hbm.at[ia_v.at[slot]], bufa.at[slot], sema)
            pltpu.async_copy(tb_hbm.at[ib_v.at[slot]], bufb.at[slot], semb)

        def drain(t, slot, sema, semb):
            off = off_of(t)
            pltpu.make_async_copy(ta_hbm.at[ia_v.at[slot]], bufa.at[slot], sema).wait()
            pltpu.sync_copy(bufa.at[slot], oa_hbm.at[pl.ds(off, b)])
            pltpu.make_async_copy(tb_hbm.at[ib_v.at[slot]], bufb.at[slot], semb).wait()
            pltpu.sync_copy(bufb.at[slot], ob_hbm.at[pl.ds(off, b)])

        start(0, 0, sa0, sb0)

        def body(i, carry):
            t0 = i * 2
            start(t0 + 1, 1, sa1, sb1)
            drain(t0, 0, sa0, sb0)

            @pl.when(i + 1 < npairs)
            def _():
                start(t0 + 2, 0, sa0, sb0)

            drain(t0 + 1, 1, sa1, sb1)
            return carry

        lax.fori_loop(0, npairs, body, 0)

    return k(tab_a, idx_a, tab_b, idx_b)


def _sc_segsum(vals, idx, n_out, n_ranges):
    e = vals.shape[0]
    rsize = -(-n_out // n_ranges)
    rsize = -(-rsize // 128) * 128
    n_pad = rsize * n_ranges
    ncks = rsize // 128
    nz = -(-ncks // _NS)
    b = 80
    ept = e // _NS
    assert e % _NS == 0 and ept % b == 0
    nchunks = ept // b
    rpc = n_ranges // 2
    mesh = plsc.VectorSubcoreMesh(core_axis_name="c", subcore_axis_name="s")

    @functools.partial(
        pl.kernel,
        out_type=[jax.ShapeDtypeStruct((rsize + 128, H), jnp.float32)
                  for _ in range(n_ranges)],
        mesh=mesh,
        scratch_types=[
            pltpu.VMEM((b,), jnp.int32),
            pltpu.VMEM((b,), jnp.int32),
            pltpu.VMEM((b, H), jnp.float32),
            pltpu.VMEM_SHARED((rsize + 128, H), jnp.float32),
        ],
    )
    def k(vals_hbm, idx_hbm, zeros_hbm, *rest):
        outs, (iraw, iadj, vbuf, acc) = rest[:n_ranges], rest[n_ranges:]
        cid = lax.axis_index("c")
        sid = lax.axis_index("s")
        ebase = sid * ept
        spread = lax.iota(jnp.int32, 16) * 8

        for r in range(rpc):
            @pl.when(sid == 0)
            def _zinit():
                pltpu.sync_copy(zeros_hbm, acc)

            plsc.subcore_barrier()
            nbase_t = (cid * rpc + r) * rsize

            def scan(t, carry):
                off = ebase + t * b
                pltpu.sync_copy(idx_hbm.at[pl.ds(off, b)], iraw)
                pltpu.sync_copy(vals_hbm.at[pl.ds(off, b)], vbuf)
                for j in range(b // 16):
                    x = iraw[pl.ds(j * 16, 16)] - nbase_t
                    oob = (x < 0) | (x >= rsize)
                    iadj[pl.ds(j * 16, 16)] = jnp.where(oob, rsize + spread + j, x)
                pltpu.sync_copy(vbuf, acc.at[iadj], add=True)
                return carry

            lax.fori_loop(0, nchunks, scan, 0)
            plsc.subcore_barrier()

            @pl.when((sid == 0) & (cid == 0))
            def _w0():
                pltpu.sync_copy(acc, outs[r])

            @pl.when((sid == 0) & (cid == 1))
            def _w1():
                pltpu.sync_copy(acc, outs[rpc + r])

            plsc.subcore_barrier()

    outs = k(vals, idx, jnp.zeros((rsize + 128, H), jnp.float32))
    return jnp.concatenate([o[:rsize] for o in outs], axis=0)


def _fold_bn(sums, n, gamma, beta):
    mean = sums[0] / n
    var = sums[1] / n - mean * mean
    s = gamma * lax.rsqrt(var + EPS)
    t = beta - mean * s
    return s, t


def kernel(v, c, e_row, e_col, e_val,
           emb_v_W0, emb_v_b0, emb_v_W1, emb_v_b1,
           emb_c_W0, emb_c_b0, emb_c_W1, emb_c_b1,
           bn_v_gamma, bn_v_beta, bn_c_gamma, bn_c_beta,
           cconv_g_W0, cconv_g_b0, cconv_g_W1, cconv_g_b1,
           cconv_f_W0, cconv_f_b0, cconv_f_W1, cconv_f_b1,
           bn_c2_gamma, bn_c2_beta,
           vconv_g_W0, vconv_g_b0, vconv_g_W1, vconv_g_b1,
           vconv_f_W0, vconv_f_b0, vconv_f_W1, vconv_f_b1,
           bn_v2_gamma, bn_v2_beta,
           tail_W0, tail_b0, tail_W1, tail_b1):
    v_n = v.shape[0]
    c_n = c.shape[0]

    v1r, v_sums = _embed(v, emb_v_W0, emb_v_b0, emb_v_W1, emb_v_b1)
    c1r, c_sums = _embed(c, emb_c_W0, emb_c_b0, emb_c_W1, emb_c_b1)
    s_v, t_v = _fold_bn(v_sums, v_n, bn_v_gamma, bn_v_beta)
    s_c, t_c = _fold_bn(c_sums, c_n, bn_c_gamma, bn_c_beta)

    wu, wv, we = cconv_g_W0[:H], cconv_g_W0[H:2 * H], cconv_g_W0[2 * H]
    wu2, wv2, we2 = vconv_g_W0[:H], vconv_g_W0[H:2 * H], vconv_g_W0[2 * H]
    tab_v = _mm(v1r, jnp.concatenate(
        [s_v[:, None] * wv, s_v[:, None] * wu2], axis=1))
    tab_c = _mm(c1r, jnp.concatenate(
        [s_c[:, None] * wu, jnp.zeros_like(wu)], axis=1))
    e_n = e_row.shape[0]
    eh = e_n // 2
    row_h = (e_row[:eh], e_row[eh:])
    col_h = (e_col[:eh], e_col[eh:])
    val_h = (e_val[:eh], e_val[eh:])
    b1 = t_c @ wu + t_v @ wv + cconv_g_b0
    gv_h, gc_h, go_h = [], [], []
    for i in range(2):
        gvi, gci = _sc_gather2(tab_v, row_h[i], tab_c, col_h[i])
        gv_h.append(gvi)
        gc_h.append(gci)
        go_h.append(_edge(gci, 0, gvi, 0, val_h[i], we, b1,
                          cconv_g_W1, cconv_g_b1))
    agg_c = (jnp.zeros((c_n, H), jnp.float32)
             .at[col_h[0]].add(go_h[0], mode='promise_in_bounds')
             + jnp.zeros((c_n, H), jnp.float32)
             .at[col_h[1]].add(go_h[1], mode='promise_in_bounds'))
    wfa, wfb = cconv_f_W0[:H], cconv_f_W0[H:]
    c2r, c2_sums = _fmlp(c1r, agg_c, s_c[:, None] * wfa, wfb,
                         t_c @ wfa + cconv_f_b0, cconv_f_W1, cconv_f_b1)
    s_c2, t_c2 = _fold_bn(c2_sums, c_n, bn_c2_gamma, bn_c2_beta)

    tab_c2 = _mm(c2r, jnp.concatenate(
        [s_c2[:, None] * wv2, jnp.zeros_like(wv2)], axis=1))
    b12 = t_v @ wu2 + t_c2 @ wv2 + vconv_g_b0
    go2_h = []
    for i in range(2):
        gc2i = _sc_gather(tab_c2, col_h[i])
        go2_h.append(_edge(gv_h[i], H, gc2i, 0, val_h[i], we2, b12,
                           vconv_g_W1, vconv_g_b1))
    agg_v = (jnp.zeros((v_n, H), jnp.float32)
             .at[row_h[0]].add(go2_h[0], mode='promise_in_bounds')
             + jnp.zeros((v_n, H), jnp.float32)
             .at[row_h[1]].add(go2_h[1], mode='promise_in_bounds'))
    wfa2, wfb2 = vconv_f_W0[:H], vconv_f_W0[H:]
    v2r, v2_sums = _fmlp(v1r, agg_v, s_v[:, None] * wfa2, wfb2,
                         t_v @ wfa2 + vconv_f_b0, vconv_f_W1, vconv_f_b1)
    s_v2, t_v2 = _fold_bn(v2_sums, v_n, bn_v2_gamma, bn_v2_beta)

    return _tail(v2r, s_v2[:, None] * tail_W0, t_v2 @ tail_W0 + tail_b0,
                 tail_W1, tail_b1)


# speedup vs baseline: 1.3109x; 1.0276x over previous
import functools

import jax
import jax.numpy as jnp
from jax import lax
from jax.experimental import pallas as pl
from jax.experimental.pallas import tpu as pltpu
from jax.experimental.pallas import tpu_sc as plsc

H = 64
EPS = 1e-5
_NC, _NS = 2, 16
_NW = _NC * _NS


def _row_tile(n):
    for t in (2000, 1000, 500, 250):
        if n % t == 0:
            return t
    return n


def _embed_body(x_ref, w0_ref, b0_ref, w1_ref, b1_ref, out_ref, sums_ref, acc_ref):
    i = pl.program_id(0)

    @pl.when(i == 0)
    def _init():
        acc_ref[...] = jnp.zeros_like(acc_ref)

    h = jnp.maximum(jnp.dot(x_ref[...], w0_ref[...],
                            preferred_element_type=jnp.float32) + b0_ref[...], 0.0)
    y = jnp.maximum(jnp.dot(h, w1_ref[...],
                            preferred_element_type=jnp.float32) + b1_ref[...], 0.0)
    out_ref[...] = y
    s = jnp.sum(y, axis=0, keepdims=True)
    sq = jnp.sum(y * y, axis=0, keepdims=True)
    acc_ref[...] += jnp.concatenate([s, sq], axis=0)

    @pl.when(i == pl.num_programs(0) - 1)
    def _fin():
        sums_ref[...] = acc_ref[...]


def _embed(x, w0, b0, w1, b1):
    n, d = x.shape
    tile = _row_tile(n)
    grid = n // tile
    out, sums = pl.pallas_call(
        _embed_body,
        grid=(grid,),
        in_specs=[
            pl.BlockSpec((tile, d), lambda i: (i, 0)),
            pl.BlockSpec((d, H), lambda i: (0, 0)),
            pl.BlockSpec((1, H), lambda i: (0, 0)),
            pl.BlockSpec((H, H), lambda i: (0, 0)),
            pl.BlockSpec((1, H), lambda i: (0, 0)),
        ],
        out_specs=[
            pl.BlockSpec((tile, H), lambda i: (i, 0)),
            pl.BlockSpec((2, H), lambda i: (0, 0)),
        ],
        out_shape=[
            jax.ShapeDtypeStruct((n, H), jnp.float32),
            jax.ShapeDtypeStruct((2, H), jnp.float32),
        ],
        scratch_shapes=[pltpu.VMEM((2, H), jnp.float32)],
    )(x, w0, b0.reshape(1, H), w1, b1.reshape(1, H))
    return out, sums


def _mm_body(x_ref, w_ref, out_ref):
    out_ref[...] = jnp.dot(x_ref[...], w_ref[...], preferred_element_type=jnp.float32)


def _mm(x, w):
    n = x.shape[0]
    m = w.shape[1]
    tile = _row_tile(n)
    return pl.pallas_call(
        _mm_body,
        grid=(n // tile,),
        in_specs=[
            pl.BlockSpec((tile, H), lambda i: (i, 0)),
            pl.BlockSpec((H, m), lambda i: (0, 0)),
        ],
        out_specs=pl.BlockSpec((tile, m), lambda i: (i, 0)),
        out_shape=jax.ShapeDtypeStruct((n, m), jnp.float32),
    )(x, w)


def _edge_body(a_off, b_off, au_ref, av_ref, ev_ref, we_ref, b1_ref, w_ref, b2_ref,
               out_ref):
    hpre = (au_ref[:, a_off:a_off + H] + av_ref[:, b_off:b_off + H]
            + ev_ref[...] * we_ref[...] + b1_ref[...])
    h = jnp.maximum(hpre, 0.0)
    g = jnp.dot(h, w_ref[...], preferred_element_type=jnp.float32) + b2_ref[...]
    out_ref[...] = jnp.maximum(g, 0.0)


def _edge(au_g, a_off, av_g, b_off, e_val, we, b1, w, b2):
    e = au_g.shape[0]
    tile = 6400
    assert e % tile == 0
    return pl.pallas_call(
        functools.partial(_edge_body, a_off, b_off),
        grid=(e // tile,),
        in_specs=[
            pl.BlockSpec((tile, au_g.shape[1]), lambda i: (i, 0)),
            pl.BlockSpec((tile, av_g.shape[1]), lambda i: (i, 0)),
            pl.BlockSpec((tile, 1), lambda i: (i, 0)),
            pl.BlockSpec((1, H), lambda i: (0, 0)),
            pl.BlockSpec((1, H), lambda i: (0, 0)),
            pl.BlockSpec((H, H), lambda i: (0, 0)),
            pl.BlockSpec((1, H), lambda i: (0, 0)),
        ],
        out_specs=pl.BlockSpec((tile, H), lambda i: (i, 0)),
        out_shape=jax.ShapeDtypeStruct((e, H), jnp.float32),
    )(au_g, av_g, e_val, we.reshape(1, H), b1.reshape(1, H), w, b2.reshape(1, H))


def _fmlp_body(u_ref, agg_ref, wa_ref, wb_ref, b0_ref, w1_ref, b1_ref,
               out_ref, sums_ref, acc_ref):
    i = pl.program_id(0)

    @pl.when(i == 0)
    def _init():
        acc_ref[...] = jnp.zeros_like(acc_ref)

    h = jnp.dot(u_ref[...], wa_ref[...], preferred_element_type=jnp.float32)
    h += jnp.dot(agg_ref[...], wb_ref[...], preferred_element_type=jnp.float32)
    h = jnp.maximum(h + b0_ref[...], 0.0)
    y = jnp.maximum(jnp.dot(h, w1_ref[...],
                            preferred_element_type=jnp.float32) + b1_ref[...], 0.0)
    out_ref[...] = y
    s = jnp.sum(y, axis=0, keepdims=True)
    sq = jnp.sum(y * y, axis=0, keepdims=True)
    acc_ref[...] += jnp.concatenate([s, sq], axis=0)

    @pl.when(i == pl.num_programs(0) - 1)
    def _fin():
        sums_ref[...] = acc_ref[...]


def _fmlp(u, agg, wa, wb, b0, w1, b1):
    n = u.shape[0]
    tile = _row_tile(n)
    out, sums = pl.pallas_call(
        _fmlp_body,
        grid=(n // tile,),
        in_specs=[
            pl.BlockSpec((tile, H), lambda i: (i, 0)),
            pl.BlockSpec((tile, H), lambda i: (i, 0)),
            pl.BlockSpec((H, H), lambda i: (0, 0)),
            pl.BlockSpec((H, H), lambda i: (0, 0)),
            pl.BlockSpec((1, H), lambda i: (0, 0)),
            pl.BlockSpec((H, H), lambda i: (0, 0)),
            pl.BlockSpec((1, H), lambda i: (0, 0)),
        ],
        out_specs=[
            pl.BlockSpec((tile, H), lambda i: (i, 0)),
            pl.BlockSpec((2, H), lambda i: (0, 0)),
        ],
        out_shape=[
            jax.ShapeDtypeStruct((n, H), jnp.float32),
            jax.ShapeDtypeStruct((2, H), jnp.float32),
        ],
        scratch_shapes=[pltpu.VMEM((2, H), jnp.float32)],
    )(u, agg, wa, wb, b0.reshape(1, H), w1, b1.reshape(1, H))
    return out, sums


def _tail_body(x_ref, w0_ref, b0_ref, w1_ref, b1_ref, out_ref):
    h = jnp.maximum(jnp.dot(x_ref[...], w0_ref[...],
                            preferred_element_type=jnp.float32) + b0_ref[...], 0.0)
    o = jnp.dot(h, w1_ref[...], preferred_element_type=jnp.float32) + b1_ref[...]
    out_ref[...] = jax.nn.sigmoid(o)


def _tail(x, w0, b0, w1, b1):
    n = x.shape[0]
    k = w1.shape[1]
    tile = _row_tile(n)
    return pl.pallas_call(
        _tail_body,
        grid=(n // tile,),
        in_specs=[
            pl.BlockSpec((tile, H), lambda i: (i, 0)),
            pl.BlockSpec((H, H), lambda i: (0, 0)),
            pl.BlockSpec((1, H), lambda i: (0, 0)),
            pl.BlockSpec((H, k), lambda i: (0, 0)),
            pl.BlockSpec((1, k), lambda i: (0, 0)),
        ],
        out_specs=pl.BlockSpec((tile, k), lambda i: (i, 0)),
        out_shape=jax.ShapeDtypeStruct((n, k), jnp.float32),
    )(x, w0, b0.reshape(1, H), w1, b1.reshape(1, k))


def _sc_gather(table, idx):
    e = idx.shape[0]
    w128 = table.shape[1]
    b = 128
    epw = e // _NW
    assert e % _NW == 0 and epw % 8 == 0
    nchunks = (epw + b - 1) // b
    nchunks += nchunks % 2
    npairs = nchunks // 2
    mesh = plsc.VectorSubcoreMesh(core_axis_name="c", subcore_axis_name="s")

    @functools.partial(
        pl.kernel,
        out_type=jax.ShapeDtypeStruct((e, w128), jnp.float32),
        mesh=mesh,
        scratch_types=[
            pltpu.VMEM((2, b), jnp.int32),
            pltpu.VMEM((2, b, w128), jnp.float32),
            pltpu.SemaphoreType.DMA, pltpu.SemaphoreType.DMA,
        ],
    )
    def k(tab_hbm, idx_hbm, out_hbm, idx_v, buf, sem0, sem1):
        wid = lax.axis_index("s") * _NC + lax.axis_index("c")
        base = wid * epw

        def off_of(t):
            return base + jnp.minimum(t * b, epw - b)

        def start(t, slot, sem):
            pltpu.sync_copy(idx_hbm.at[pl.ds(off_of(t), b)], idx_v.at[slot])
            pltpu.async_copy(tab_hbm.at[idx_v.at[slot]], buf.at[slot], sem)

        def drain(t, slot, sem):
            pltpu.make_async_copy(tab_hbm.at[idx_v.at[slot]], buf.at[slot], sem).wait()
            pltpu.sync_copy(buf.at[slot], out_hbm.at[pl.ds(off_of(t), b)])

        start(0, 0, sem0)

        def body(i, carry):
            t0 = i * 2
            start(t0 + 1, 1, sem1)
            drain(t0, 0, sem0)

            @pl.when(i + 1 < npairs)
            def _():
                start(t0 + 2, 0, sem0)

            drain(t0 + 1, 1, sem1)
            return carry

        lax.fori_loop(0, npairs, body, 0)

    return k(table, idx)


def _sc_gather2(tab_a, idx_a, tab_b, idx_b):
    e = idx_a.shape[0]
    wa, wb = tab_a.shape[1], tab_b.shape[1]
    b = 128
    epw = e // _NW
    assert e % _NW == 0 and epw % 8 == 0
    nchunks = (epw + b - 1) // b
    nchunks += nchunks % 2
    npairs = nchunks // 2
    mesh = plsc.VectorSubcoreMesh(core_axis_name="c", subcore_axis_name="s")

    @functools.partial(
        pl.kernel,
        out_type=[jax.ShapeDtypeStruct((e, wa), jnp.float32),
                  jax.ShapeDtypeStruct((e, wb), jnp.float32)],
        mesh=mesh,
        scratch_types=[
            pltpu.VMEM((2, b), jnp.int32), pltpu.VMEM((2, b), jnp.int32),
            pltpu.VMEM((2, b, wa), jnp.float32), pltpu.VMEM((2, b, wb), jnp.float32),
            pltpu.SemaphoreType.DMA, pltpu.SemaphoreType.DMA,
            pltpu.SemaphoreType.DMA, pltpu.SemaphoreType.DMA,
        ],
    )
    def k(ta_hbm, ia_hbm, tb_hbm, ib_hbm, oa_hbm, ob_hbm,
          ia_v, ib_v, bufa, bufb, sa0, sa1, sb0, sb1):
        wid = lax.axis_index("s") * _NC + lax.axis_index("c")
        base = wid * epw

        def off_of(t):
            return base + jnp.minimum(t * b, epw - b)

        def start(t, slot, sema, semb):
            off = off_of(t)
            pltpu.sync_copy(ia_hbm.at[pl.ds(off, b)], ia_v.at[slot])
            pltpu.sync_copy(ib_hbm.at[pl.ds(off, b)], ib_v.at[slot])
            pltpu.async_copy(ta_hbm.at[ia_v.at[slot]], bufa.at[slot], sema)
            pltpu.async_copy(tb_hbm.at[ib_v.at[slot]], bufb.at[slot], semb)

        def drain(t, slot, sema, semb):
            off = off_of(t)
            pltpu.make_async_copy(ta_hbm.at[ia_v.at[slot]], bufa.at[slot], sema).wait()
            pltpu.sync_copy(bufa.at[slot], oa_hbm.at[pl.ds(off, b)])
            pltpu.make_async_copy(tb_hbm.at[ib_v.at[slot]], bufb.at[slot], semb).wait()
            pltpu.sync_copy(bufb.at[slot], ob_hbm.at[pl.ds(off, b)])

        start(0, 0, sa0, sb0)

        def body(i, carry):
            t0 = i * 2
            start(t0 + 1, 1, sa1, sb1)
            drain(t0, 0, sa0, sb0)

            @pl.when(i + 1 < npairs)
            def _():
                start(t0 + 2, 0, sa0, sb0)

            drain(t0 + 1, 1, sa1, sb1)
            return carry

        lax.fori_loop(0, npairs, body, 0)

    return k(tab_a, idx_a, tab_b, idx_b)


def _sc_segsum(vals, idx, n_out, n_ranges):
    e = vals.shape[0]
    rsize = -(-n_out // n_ranges)
    rsize = -(-rsize // 128) * 128
    n_pad = rsize * n_ranges
    ncks = rsize // 128
    nz = -(-ncks // _NS)
    b = 80
    ept = e // _NS
    assert e % _NS == 0 and ept % b == 0
    nchunks = ept // b
    rpc = n_ranges // 2
    mesh = plsc.VectorSubcoreMesh(core_axis_name="c", subcore_axis_name="s")

    @functools.partial(
        pl.kernel,
        out_type=[jax.ShapeDtypeStruct((rsize + 128, H), jnp.float32)
                  for _ in range(n_ranges)],
        mesh=mesh,
        scratch_types=[
            pltpu.VMEM((b,), jnp.int32),
            pltpu.VMEM((b,), jnp.int32),
            pltpu.VMEM((b, H), jnp.float32),
            pltpu.VMEM_SHARED((rsize + 128, H), jnp.float32),
        ],
    )
    def k(vals_hbm, idx_hbm, zeros_hbm, *rest):
        outs, (iraw, iadj, vbuf, acc) = rest[:n_ranges], rest[n_ranges:]
        cid = lax.axis_index("c")
        sid = lax.axis_index("s")
        ebase = sid * ept
        spread = lax.iota(jnp.int32, 16) * 8

        for r in range(rpc):
            @pl.when(sid == 0)
            def _zinit():
                pltpu.sync_copy(zeros_hbm, acc)

            plsc.subcore_barrier()
            nbase_t = (cid * rpc + r) * rsize

            def scan(t, carry):
                off = ebase + t * b
                pltpu.sync_copy(idx_hbm.at[pl.ds(off, b)], iraw)
                pltpu.sync_copy(vals_hbm.at[pl.ds(off, b)], vbuf)
                for j in range(b // 16):
                    x = iraw[pl.ds(j * 16, 16)] - nbase_t
                    oob = (x < 0) | (x >= rsize)
                    iadj[pl.ds(j * 16, 16)] = jnp.where(oob, rsize + spread + j, x)
                pltpu.sync_copy(vbuf, acc.at[iadj], add=True)
                return carry

            lax.fori_loop(0, nchunks, scan, 0)
            plsc.subcore_barrier()

            @pl.when((sid == 0) & (cid == 0))
            def _w0():
                pltpu.sync_copy(acc, outs[r])

            @pl.when((sid == 0) & (cid == 1))
            def _w1():
                pltpu.sync_copy(acc, outs[rpc + r])

            plsc.subcore_barrier()

    outs = k(vals, idx, jnp.zeros((rsize + 128, H), jnp.float32))
    return jnp.concatenate([o[:rsize] for o in outs], axis=0)


def _fold_bn(sums, n, gamma, beta):
    mean = sums[0] / n
    var = sums[1] / n - mean * mean
    s = gamma * lax.rsqrt(var + EPS)
    t = beta - mean * s
    return s, t


def kernel(v, c, e_row, e_col, e_val,
           emb_v_W0, emb_v_b0, emb_v_W1, emb_v_b1,
           emb_c_W0, emb_c_b0, emb_c_W1, emb_c_b1,
           bn_v_gamma, bn_v_beta, bn_c_gamma, bn_c_beta,
           cconv_g_W0, cconv_g_b0, cconv_g_W1, cconv_g_b1,
           cconv_f_W0, cconv_f_b0, cconv_f_W1, cconv_f_b1,
           bn_c2_gamma, bn_c2_beta,
           vconv_g_W0, vconv_g_b0, vconv_g_W1, vconv_g_b1,
           vconv_f_W0, vconv_f_b0, vconv_f_W1, vconv_f_b1,
           bn_v2_gamma, bn_v2_beta,
           tail_W0, tail_b0, tail_W1, tail_b1):
    v_n = v.shape[0]
    c_n = c.shape[0]

    v1r, v_sums = _embed(v, emb_v_W0, emb_v_b0, emb_v_W1, emb_v_b1)
    c1r, c_sums = _embed(c, emb_c_W0, emb_c_b0, emb_c_W1, emb_c_b1)
    s_v, t_v = _fold_bn(v_sums, v_n, bn_v_gamma, bn_v_beta)
    s_c, t_c = _fold_bn(c_sums, c_n, bn_c_gamma, bn_c_beta)

    wu, wv, we = cconv_g_W0[:H], cconv_g_W0[H:2 * H], cconv_g_W0[2 * H]
    wu2, wv2, we2 = vconv_g_W0[:H], vconv_g_W0[H:2 * H], vconv_g_W0[2 * H]
    tab_v = _mm(v1r, jnp.concatenate(
        [s_v[:, None] * wv, s_v[:, None] * wu2], axis=1))
    tab_c = _mm(c1r, jnp.concatenate(
        [s_c[:, None] * wu, jnp.zeros_like(wu)], axis=1))
    e_n = e_row.shape[0]
    nsp = 5
    eh = e_n // nsp
    row_h = tuple(e_row[i * eh:(i + 1) * eh] for i in range(nsp))
    col_h = tuple(e_col[i * eh:(i + 1) * eh] for i in range(nsp))
    val_h = tuple(e_val[i * eh:(i + 1) * eh] for i in range(nsp))
    b1 = t_c @ wu + t_v @ wv + cconv_g_b0
    gv_h, gc_h, go_h = [], [], []
    for i in range(nsp):
        gvi, gci = _sc_gather2(tab_v, row_h[i], tab_c, col_h[i])
        gv_h.append(gvi)
        gc_h.append(gci)
        go_h.append(_edge(gci, 0, gvi, 0, val_h[i], we, b1,
                          cconv_g_W1, cconv_g_b1))
    agg_c = sum(jnp.zeros((c_n, H), jnp.float32)
                .at[col_h[i]].add(go_h[i], mode='promise_in_bounds')
                for i in range(nsp))
    wfa, wfb = cconv_f_W0[:H], cconv_f_W0[H:]
    c2r, c2_sums = _fmlp(c1r, agg_c, s_c[:, None] * wfa, wfb,
                         t_c @ wfa + cconv_f_b0, cconv_f_W1, cconv_f_b1)
    s_c2, t_c2 = _fold_bn(c2_sums, c_n, bn_c2_gamma, bn_c2_beta)

    tab_c2 = _mm(c2r, jnp.concatenate(
        [s_c2[:, None] * wv2, jnp.zeros_like(wv2)], axis=1))
    b12 = t_v @ wu2 + t_c2 @ wv2 + vconv_g_b0
    go2_h = []
    for i in range(nsp):
        gc2i = _sc_gather(tab_c2, col_h[i])
        go2_h.append(_edge(gv_h[i], H, gc2i, 0, val_h[i], we2, b12,
                           vconv_g_W1, vconv_g_b1))
    agg_v = sum(jnp.zeros((v_n, H), jnp.float32)
                .at[row_h[i]].add(go2_h[i], mode='promise_in_bounds')
                for i in range(nsp))
    wfa2, wfb2 = vconv_f_W0[:H], vconv_f_W0[H:]
    v2r, v2_sums = _fmlp(v1r, agg_v, s_v[:, None] * wfa2, wfb2,
                         t_v @ wfa2 + vconv_f_b0, vconv_f_W1, vconv_f_b1)
    s_v2, t_v2 = _fold_bn(v2_sums, v_n, bn_v2_gamma, bn_v2_beta)

    return _tail(v2r, s_v2[:, None] * tail_W0, t_v2 @ tail_W0 + tail_b0,
                 tail_W1, tail_b1)
